# baseline (speedup 1.0000x reference)
import functools

import jax
import jax.numpy as jnp
from jax import lax
from jax.experimental import pallas as pl
from jax.experimental.pallas import tpu as pltpu
from jax.experimental.pallas import tpu_sc as plsc

NUM_CLASSES = 1024
FEAT_DIM = 256
BATCH = 4096
K = 15
BLK = 512
PAD = 1 << 30
E_MIN = 0.9 ** 10
NW = 32
ROWS_PER_TILE = NUM_CLASSES // NW


def _sqdist(x, cvs):
    x2 = jnp.sum(x * x, axis=1, keepdims=True)
    c2 = jnp.sum(cvs * cvs, axis=1)[None, :]
    xc = lax.dot_general(x, cvs, (((1,), (1,)), ((), ())),
                         preferred_element_type=jnp.float32)
    return x2 + c2 - 2.0 * xc


def _topk_body(x_ref, cvs_ref, topk_ref, xcvs_ref):
    x = x_ref[...]
    cvs = cvs_ref[...]
    sq = _sqdist(x, cvs)
    xcvs_ref[...] = jnp.maximum(sq, 1e-12)
    NCH = 8
    CW = NUM_CLASSES // NCH
    colw = lax.broadcasted_iota(jnp.int32, (BLK, CW), 1)
    kcol16 = lax.broadcasted_iota(jnp.int32, (BLK, 16), 1)
    cand_v, cand_i = [], []
    for c in range(NCH):
        w = sq[:, c * CW:(c + 1) * CW]
        vacc = jnp.full((BLK, 16), jnp.inf, jnp.float32)
        iacc = jnp.full((BLK, 16), NUM_CLASSES, jnp.int32)
        for k in range(K):
            m = jnp.min(w, axis=1, keepdims=True)
            idx = jnp.min(jnp.where(w == m, colw, CW), axis=1, keepdims=True)
            vacc = jnp.where(kcol16 == k, m, vacc)
            iacc = jnp.where(kcol16 == k, idx + c * CW, iacc)
            w = jnp.where(colw == idx, jnp.inf, w)
        cand_v.append(vacc)
        cand_i.append(iacc)
    vals = jnp.concatenate(cand_v, axis=1)
    gidx = jnp.concatenate(cand_i, axis=1)
    out = jnp.full((BLK, 16), PAD, jnp.int32)
    idx0 = None
    for k in range(K):
        m = jnp.min(vals, axis=1, keepdims=True)
        gi = jnp.min(jnp.where(vals == m, gidx, NUM_CLASSES), axis=1,
                     keepdims=True)
        if k == 0:
            idx0 = gi
        out = jnp.where(kcol16 == k, idx0 * NUM_CLASSES + gi, out)
        vals = jnp.where((vals == m) & (gidx == gi), jnp.inf, vals)
    topk_ref[...] = out


def _stage_a(x, cvs):
    return pl.pallas_call(
        _topk_body,
        grid=(BATCH // BLK,),
        in_specs=[
            pl.BlockSpec((BLK, FEAT_DIM), lambda i: (i, 0)),
            pl.BlockSpec((NUM_CLASSES, FEAT_DIM), lambda i: (0, 0)),
        ],
        out_specs=[
            pl.BlockSpec((BLK, 16), lambda i: (i, 0)),
            pl.BlockSpec((BLK, NUM_CLASSES), lambda i: (i, 0)),
        ],
        out_shape=[
            jax.ShapeDtypeStruct((BATCH, 16), jnp.int32),
            jax.ShapeDtypeStruct((BATCH, NUM_CLASSES), jnp.float32),
        ],
    )(x, cvs)


def _sc_scatter_body(pairs_hbm, cce_hbm, pairs_v, buf_v):
    wid = lax.axis_index("c") * 16 + lax.axis_index("s")
    base_row = wid * ROWS_PER_TILE
    pltpu.sync_copy(pairs_hbm, pairs_v)

    zeros16 = jnp.zeros((16,), jnp.float32)

    @plsc.parallel_loop(0, ROWS_PER_TILE * NUM_CLASSES // 16, unroll=8)
    def _zero(i):
        buf_v[pl.ds(i * 16, 16)] = zeros16

    ones = jnp.ones((16,), jnp.float32)
    lo = base_row * NUM_CLASSES
    span = ROWS_PER_TILE * NUM_CLASSES

    @plsc.parallel_loop(0, BATCH, unroll=8)
    def _scatter(s):
        vec = pairs_v[pl.ds(s * 16, 16)]
        local = vec - lo
        mask = plsc.bitcast(local, jnp.uint32) < jnp.uint32(span)
        safe = jnp.where(mask, local, 0)
        plsc.addupdate_scatter(buf_v, [safe], ones, mask=mask)
    pltpu.sync_copy(
        buf_v,
        cce_hbm.at[pl.ds(base_row * NUM_CLASSES, ROWS_PER_TILE * NUM_CLASSES)])


@functools.lru_cache(maxsize=None)
def _make_sc_scatter():
    mesh = plsc.VectorSubcoreMesh(core_axis_name="c", subcore_axis_name="s")
    return pl.kernel(
        _sc_scatter_body,
        mesh=mesh,
        out_type=jax.ShapeDtypeStruct((NUM_CLASSES * NUM_CLASSES,),
                                      jnp.float32),
        scratch_types=[
            pltpu.VMEM((BATCH * 16,), jnp.int32),
            pltpu.VMEM((ROWS_PER_TILE * NUM_CLASSES,), jnp.float32),
        ],
        compiler_params=pltpu.CompilerParams(needs_layout_passes=False),
    )


_GB = BATCH // NW
_HALF = _GB // 2


def _sc_gather_body(cce_hbm, labels_hbm, out_hbm, idx_v, rows_v, sem):
    wid = lax.axis_index("c") * 16 + lax.axis_index("s")
    base = wid * _HALF
    pltpu.sync_copy(labels_hbm.at[pl.ds(base, _HALF)], idx_v)
    pltpu.async_copy(cce_hbm.at[idx_v], rows_v, sem).wait()
    pltpu.sync_copy(rows_v, out_hbm.at[pl.ds(base, _HALF)])


@functools.lru_cache(maxsize=None)
def _make_sc_gather():
    mesh = plsc.VectorSubcoreMesh(core_axis_name="c", subcore_axis_name="s")
    return pl.kernel(
        _sc_gather_body,
        mesh=mesh,
        out_type=jax.ShapeDtypeStruct((BATCH // 2, NUM_CLASSES), jnp.float32),
        scratch_types=[
            pltpu.VMEM((_HALF,), jnp.int32),
            pltpu.VMEM((_HALF, NUM_CLASSES), jnp.float32),
            pltpu.SemaphoreType.DMA,
        ],
        compiler_params=pltpu.CompilerParams(needs_layout_passes=False),
    )


def _loss_body(xcvs_ref, gath_ref, lab_ref, out_ref):
    i = pl.program_id(0)
    xc = xcvs_ref[...]
    g = gath_ref[...]
    lab = lab_ref[...]
    colid = lax.broadcasted_iota(jnp.int32, (BLK, NUM_CLASSES), 1)
    scale = jnp.sum(jnp.where(colid == lab, xc, 0.0), axis=1)
    visited = jnp.sum(g, axis=1, keepdims=True) / 15.0
    enc = visited - g
    enc = enc * (enc > 0)
    eye = (colid == lab).astype(jnp.float32)
    exist = (g > 0.0).astype(jnp.float32)
    edges = jnp.maximum(eye, exist) * jnp.power(jnp.float32(0.9), enc)
    edges = edges * (1.0 - (edges < E_MIN).astype(jnp.float32))
    gm = ((edges > 0.0) & (colid != lab)).astype(jnp.float32)
    d_neg = xc * gm
    e = jnp.exp(-0.001 * d_neg) * (d_neg > 0.0)
    s = jnp.sum(e, axis=1)
    t = jnp.sum(e * d_neg, axis=1)
    mu = scale - jnp.where(s > 0.0, t / s, 0.0)
    part = jnp.sum(jnp.maximum(mu, 0.0))
    prev = jnp.where(i == 0, 0.0, out_ref[0, 0])
    out_ref[...] = jnp.reshape(prev + part, (1, 1))


def _stage_e(x_cvs, gath, labels2d, half):
    off = half * (BATCH // 2 // BLK)
    return pl.pallas_call(
        _loss_body,
        grid=(BATCH // 2 // BLK,),
        in_specs=[
            pl.BlockSpec((BLK, NUM_CLASSES), lambda i: (i + off, 0)),
            pl.BlockSpec((BLK, NUM_CLASSES), lambda i: (i, 0)),
            pl.BlockSpec((BLK, 1), lambda i: (i + off, 0)),
        ],
        out_specs=pl.BlockSpec((1, 1), lambda i: (0, 0)),
        out_shape=jax.ShapeDtypeStruct((1, 1), jnp.float32),
    )(x_cvs, gath, labels2d)


def kernel(x, labels, cvs):
    labels = labels.astype(jnp.int32)
    labels2d = labels.reshape(BATCH, 1)
    topk, x_cvs = _stage_a(x, cvs)
    pairs = topk.reshape(-1)
    cce = _make_sc_scatter()(pairs).reshape(NUM_CLASSES, NUM_CLASSES)
    gather = _make_sc_gather()
    half = BATCH // 2
    gath0 = gather(cce, labels[:half])
    gath1 = gather(cce, labels[half:])
    lossmat0 = _stage_e(x_cvs, gath0, labels2d, 0)
    lossmat1 = _stage_e(x_cvs, gath1, labels2d, 1)
    return (lossmat0[0, 0] + lossmat1[0, 0]) / jnp.float32(BATCH)


# speedup vs baseline: 3.3414x; 3.3414x over previous
import functools

import jax
import jax.numpy as jnp
from jax import lax
from jax.experimental import pallas as pl
from jax.experimental.pallas import tpu as pltpu
from jax.experimental.pallas import tpu_sc as plsc

NUM_CLASSES = 1024
FEAT_DIM = 256
BATCH = 4096
K = 15
BLK = 512
PAD = 1 << 30
E_MIN = 0.9 ** 10
NW = 32
ROWS_PER_TILE = NUM_CLASSES // NW


def _sqdist(x, cvs):
    x2 = jnp.sum(x * x, axis=1, keepdims=True)
    c2 = jnp.sum(cvs * cvs, axis=1)[None, :]
    xc = lax.dot_general(x, cvs, (((1,), (1,)), ((), ())),
                         preferred_element_type=jnp.float32)
    return x2 + c2 - 2.0 * xc


def _topk_body(x_ref, cvs_ref, topk_ref, xcvs_ref):
    x = x_ref[...]
    cvs = cvs_ref[...]
    sq = _sqdist(x, cvs)
    xcvs_ref[...] = jnp.maximum(sq, 1e-12)
    colid = lax.broadcasted_iota(jnp.int32, (BLK, NUM_CLASSES), 1)
    kcol = lax.broadcasted_iota(jnp.int32, (BLK, 16), 1)
    w = sq
    out = jnp.full((BLK, 16), PAD, jnp.int32)
    idx0 = None
    for k in range(K):
        m = jnp.min(w, axis=1, keepdims=True)
        idx = jnp.min(jnp.where(w == m, colid, NUM_CLASSES), axis=1,
                      keepdims=True)
        if k == 0:
            idx0 = idx
        out = jnp.where(kcol == k, idx0 * NUM_CLASSES + idx, out)
        w = jnp.where(colid == idx, jnp.inf, w)
    topk_ref[...] = out


def _stage_a(x, cvs):
    return pl.pallas_call(
        _topk_body,
        grid=(BATCH // BLK,),
        in_specs=[
            pl.BlockSpec((BLK, FEAT_DIM), lambda i: (i, 0)),
            pl.BlockSpec((NUM_CLASSES, FEAT_DIM), lambda i: (0, 0)),
        ],
        out_specs=[
            pl.BlockSpec((BLK, 16), lambda i: (i, 0)),
            pl.BlockSpec((BLK, NUM_CLASSES), lambda i: (i, 0)),
        ],
        out_shape=[
            jax.ShapeDtypeStruct((BATCH, 16), jnp.int32),
            jax.ShapeDtypeStruct((BATCH, NUM_CLASSES), jnp.float32),
        ],
    )(x, cvs)


def _sc_scatter_body(pairs_hbm, cce_hbm, pairs_v, buf_v):
    wid = lax.axis_index("c") * 16 + lax.axis_index("s")
    base_row = wid * ROWS_PER_TILE
    pltpu.sync_copy(pairs_hbm, pairs_v)

    zeros16 = jnp.zeros((16,), jnp.float32)

    @plsc.parallel_loop(0, ROWS_PER_TILE * NUM_CLASSES // 16, unroll=8)
    def _zero(i):
        buf_v[pl.ds(i * 16, 16)] = zeros16

    ones = jnp.ones((16,), jnp.float32)
    lo = base_row * NUM_CLASSES
    span = ROWS_PER_TILE * NUM_CLASSES

    @plsc.parallel_loop(0, BATCH, unroll=8)
    def _scatter(s):
        vec = pairs_v[pl.ds(s * 16, 16)]
        local = vec - lo
        mask = plsc.bitcast(local, jnp.uint32) < jnp.uint32(span)
        safe = jnp.where(mask, local, 0)
        plsc.addupdate_scatter(buf_v, [safe], ones, mask=mask)
    pltpu.sync_copy(
        buf_v,
        cce_hbm.at[pl.ds(base_row * NUM_CLASSES, ROWS_PER_TILE * NUM_CLASSES)])


@functools.lru_cache(maxsize=None)
def _make_sc_scatter():
    mesh = plsc.VectorSubcoreMesh(core_axis_name="c", subcore_axis_name="s")
    return pl.kernel(
        _sc_scatter_body,
        mesh=mesh,
        out_type=jax.ShapeDtypeStruct((NUM_CLASSES * NUM_CLASSES,),
                                      jnp.float32),
        scratch_types=[
            pltpu.VMEM((BATCH * 16,), jnp.int32),
            pltpu.VMEM((ROWS_PER_TILE * NUM_CLASSES,), jnp.float32),
        ],
        compiler_params=pltpu.CompilerParams(needs_layout_passes=False),
    )


_GB = BATCH // NW
_HALF = _GB // 2


def _sc_gather_body(cce_hbm, labels_hbm, out_hbm, idx_v, rows_v, sem):
    wid = lax.axis_index("c") * 16 + lax.axis_index("s")
    base = wid * _HALF
    pltpu.sync_copy(labels_hbm.at[pl.ds(base, _HALF)], idx_v)
    pltpu.async_copy(cce_hbm.at[idx_v], rows_v, sem).wait()
    pltpu.sync_copy(rows_v, out_hbm.at[pl.ds(base, _HALF)])


@functools.lru_cache(maxsize=None)
def _make_sc_gather():
    mesh = plsc.VectorSubcoreMesh(core_axis_name="c", subcore_axis_name="s")
    return pl.kernel(
        _sc_gather_body,
        mesh=mesh,
        out_type=jax.ShapeDtypeStruct((BATCH // 2, NUM_CLASSES), jnp.float32),
        scratch_types=[
            pltpu.VMEM((_HALF,), jnp.int32),
            pltpu.VMEM((_HALF, NUM_CLASSES), jnp.float32),
            pltpu.SemaphoreType.DMA,
        ],
        compiler_params=pltpu.CompilerParams(needs_layout_passes=False),
    )


def _loss_body(xcvs_ref, gath_ref, lab_ref, out_ref):
    i = pl.program_id(0)
    xc = xcvs_ref[...]
    g = gath_ref[...]
    lab = lab_ref[...]
    colid = lax.broadcasted_iota(jnp.int32, (BLK, NUM_CLASSES), 1)
    scale = jnp.sum(jnp.where(colid == lab, xc, 0.0), axis=1)
    visited = jnp.sum(g, axis=1, keepdims=True) / 15.0
    enc = visited - g
    enc = enc * (enc > 0)
    eye = (colid == lab).astype(jnp.float32)
    exist = (g > 0.0).astype(jnp.float32)
    edges = jnp.maximum(eye, exist) * jnp.power(jnp.float32(0.9), enc)
    edges = edges * (1.0 - (edges < E_MIN).astype(jnp.float32))
    gm = ((edges > 0.0) & (colid != lab)).astype(jnp.float32)
    d_neg = xc * gm
    e = jnp.exp(-0.001 * d_neg) * (d_neg > 0.0)
    s = jnp.sum(e, axis=1)
    t = jnp.sum(e * d_neg, axis=1)
    mu = scale - jnp.where(s > 0.0, t / s, 0.0)
    part = jnp.sum(jnp.maximum(mu, 0.0))
    prev = jnp.where(i == 0, 0.0, out_ref[0, 0])
    out_ref[...] = jnp.reshape(prev + part, (1, 1))


def _stage_e(x_cvs, gath, labels2d, half):
    off = half * (BATCH // 2 // BLK)
    return pl.pallas_call(
        _loss_body,
        grid=(BATCH // 2 // BLK,),
        in_specs=[
            pl.BlockSpec((BLK, NUM_CLASSES), lambda i: (i + off, 0)),
            pl.BlockSpec((BLK, NUM_CLASSES), lambda i: (i, 0)),
            pl.BlockSpec((BLK, 1), lambda i: (i + off, 0)),
        ],
        out_specs=pl.BlockSpec((1, 1), lambda i: (0, 0)),
        out_shape=jax.ShapeDtypeStruct((1, 1), jnp.float32),
    )(x_cvs, gath, labels2d)


def kernel(x, labels, cvs):
    labels = labels.astype(jnp.int32)
    labels2d = labels.reshape(BATCH, 1)
    topk, x_cvs = _stage_a(x, cvs)
    pairs = topk.reshape(-1)
    cce = _make_sc_scatter()(pairs).reshape(NUM_CLASSES, NUM_CLASSES)
    gather = _make_sc_gather()
    half = BATCH // 2
    gath0 = gather(cce, labels[:half])
    gath1 = gather(cce, labels[half:])
    lossmat0 = _stage_e(x_cvs, gath0, labels2d, 0)
    lossmat1 = _stage_e(x_cvs, gath1, labels2d, 1)
    return (lossmat0[0, 0] + lossmat1[0, 0]) / jnp.float32(BATCH)


# speedup vs baseline: 3.3668x; 1.0076x over previous
import functools

import jax
import jax.numpy as jnp
from jax import lax
from jax.experimental import pallas as pl
from jax.experimental.pallas import tpu as pltpu
from jax.experimental.pallas import tpu_sc as plsc

NUM_CLASSES = 1024
FEAT_DIM = 256
BATCH = 4096
K = 15
BLK = 512
PAD = 1 << 30
E_MIN = 0.9 ** 10
NW = 32
ROWS_PER_TILE = NUM_CLASSES // NW


def _sqdist(x, cvs):
    x2 = jnp.sum(x * x, axis=1, keepdims=True)
    c2 = jnp.sum(cvs * cvs, axis=1)[None, :]
    xc = lax.dot_general(x, cvs, (((1,), (1,)), ((), ())),
                         preferred_element_type=jnp.float32)
    return x2 + c2 - 2.0 * xc


def _topk_body(x_ref, cvs_ref, topk_ref, xcvs_ref):
    x = x_ref[...]
    cvs = cvs_ref[...]
    sq = _sqdist(x, cvs)
    xcvs_ref[...] = jnp.maximum(sq, 1e-12)
    colid = lax.broadcasted_iota(jnp.int32, (BLK, NUM_CLASSES), 1)
    kcol = lax.broadcasted_iota(jnp.int32, (BLK, 16), 1)
    w = sq
    out = jnp.full((BLK, 16), PAD, jnp.int32)
    idx0 = None
    for k in range(K):
        m = jnp.min(w, axis=1, keepdims=True)
        idx = jnp.min(jnp.where(w == m, colid, NUM_CLASSES), axis=1,
                      keepdims=True)
        if k == 0:
            idx0 = idx
        out = jnp.where(kcol == k, idx0 * NUM_CLASSES + idx, out)
        w = jnp.where(colid == idx, jnp.inf, w)
    topk_ref[...] = out


def _stage_a(x, cvs):
    return pl.pallas_call(
        _topk_body,
        grid=(BATCH // BLK,),
        in_specs=[
            pl.BlockSpec((BLK, FEAT_DIM), lambda i: (i, 0)),
            pl.BlockSpec((NUM_CLASSES, FEAT_DIM), lambda i: (0, 0)),
        ],
        out_specs=[
            pl.BlockSpec((BLK, 16), lambda i: (i, 0)),
            pl.BlockSpec((BLK, NUM_CLASSES), lambda i: (i, 0)),
        ],
        out_shape=[
            jax.ShapeDtypeStruct((BATCH, 16), jnp.int32),
            jax.ShapeDtypeStruct((BATCH, NUM_CLASSES), jnp.float32),
        ],
    )(x, cvs)


def _sc_scatter_body(pairs_hbm, cce_hbm, pairs_v, buf_v, sem):
    wid = lax.axis_index("c") * 16 + lax.axis_index("s")
    base_row = wid * ROWS_PER_TILE
    cp = pltpu.async_copy(pairs_hbm, pairs_v, sem)

    zeros16 = jnp.zeros((16,), jnp.float32)

    @plsc.parallel_loop(0, ROWS_PER_TILE * NUM_CLASSES // 16, unroll=8)
    def _zero(i):
        buf_v[pl.ds(i * 16, 16)] = zeros16

    cp.wait()

    ones = jnp.ones((16,), jnp.float32)
    lo = base_row * NUM_CLASSES
    span = ROWS_PER_TILE * NUM_CLASSES

    @plsc.parallel_loop(0, BATCH, unroll=16)
    def _scatter(s):
        vec = pairs_v[pl.ds(s * 16, 16)]
        local = vec - lo
        mask = plsc.bitcast(local, jnp.uint32) < jnp.uint32(span)
        safe = jnp.where(mask, local, 0)
        plsc.addupdate_scatter(buf_v, [safe], ones, mask=mask)
    pltpu.sync_copy(
        buf_v,
        cce_hbm.at[pl.ds(base_row * NUM_CLASSES, ROWS_PER_TILE * NUM_CLASSES)])


@functools.lru_cache(maxsize=None)
def _make_sc_scatter():
    mesh = plsc.VectorSubcoreMesh(core_axis_name="c", subcore_axis_name="s")
    return pl.kernel(
        _sc_scatter_body,
        mesh=mesh,
        out_type=jax.ShapeDtypeStruct((NUM_CLASSES * NUM_CLASSES,),
                                      jnp.float32),
        scratch_types=[
            pltpu.VMEM((BATCH * 16,), jnp.int32),
            pltpu.VMEM((ROWS_PER_TILE * NUM_CLASSES,), jnp.float32),
            pltpu.SemaphoreType.DMA,
        ],
        compiler_params=pltpu.CompilerParams(needs_layout_passes=False),
    )


_GB = BATCH // NW
_HALF = _GB // 2


def _sc_gather_body(cce_hbm, labels_hbm, out_hbm, idx_v, rows_v, sem):
    wid = lax.axis_index("c") * 16 + lax.axis_index("s")
    base = wid * _HALF
    pltpu.sync_copy(labels_hbm.at[pl.ds(base, _HALF)], idx_v)
    pltpu.async_copy(cce_hbm.at[idx_v], rows_v, sem).wait()
    pltpu.sync_copy(rows_v, out_hbm.at[pl.ds(base, _HALF)])


@functools.lru_cache(maxsize=None)
def _make_sc_gather():
    mesh = plsc.VectorSubcoreMesh(core_axis_name="c", subcore_axis_name="s")
    return pl.kernel(
        _sc_gather_body,
        mesh=mesh,
        out_type=jax.ShapeDtypeStruct((BATCH // 2, NUM_CLASSES), jnp.float32),
        scratch_types=[
            pltpu.VMEM((_HALF,), jnp.int32),
            pltpu.VMEM((_HALF, NUM_CLASSES), jnp.float32),
            pltpu.SemaphoreType.DMA,
        ],
        compiler_params=pltpu.CompilerParams(needs_layout_passes=False),
    )


def _loss_body(xcvs_ref, gath_ref, lab_ref, out_ref):
    i = pl.program_id(0)
    xc = xcvs_ref[...]
    g = gath_ref[...]
    lab = lab_ref[...]
    colid = lax.broadcasted_iota(jnp.int32, (BLK, NUM_CLASSES), 1)
    scale = jnp.sum(jnp.where(colid == lab, xc, 0.0), axis=1)
    visited = jnp.sum(g, axis=1, keepdims=True) / 15.0
    enc = visited - g
    enc = enc * (enc > 0)
    eye = (colid == lab).astype(jnp.float32)
    exist = (g > 0.0).astype(jnp.float32)
    edges = jnp.maximum(eye, exist) * jnp.power(jnp.float32(0.9), enc)
    edges = edges * (1.0 - (edges < E_MIN).astype(jnp.float32))
    gm = ((edges > 0.0) & (colid != lab)).astype(jnp.float32)
    d_neg = xc * gm
    e = jnp.exp(-0.001 * d_neg) * (d_neg > 0.0)
    s = jnp.sum(e, axis=1)
    t = jnp.sum(e * d_neg, axis=1)
    mu = scale - jnp.where(s > 0.0, t / s, 0.0)
    part = jnp.sum(jnp.maximum(mu, 0.0))
    prev = jnp.where(i == 0, 0.0, out_ref[0, 0])
    out_ref[...] = jnp.reshape(prev + part, (1, 1))


def _stage_e(x_cvs, gath, labels2d, half):
    off = half * (BATCH // 2 // BLK)
    return pl.pallas_call(
        _loss_body,
        grid=(BATCH // 2 // BLK,),
        in_specs=[
            pl.BlockSpec((BLK, NUM_CLASSES), lambda i: (i + off, 0)),
            pl.BlockSpec((BLK, NUM_CLASSES), lambda i: (i, 0)),
            pl.BlockSpec((BLK, 1), lambda i: (i + off, 0)),
        ],
        out_specs=pl.BlockSpec((1, 1), lambda i: (0, 0)),
        out_shape=jax.ShapeDtypeStruct((1, 1), jnp.float32),
    )(x_cvs, gath, labels2d)


def kernel(x, labels, cvs):
    labels = labels.astype(jnp.int32)
    labels2d = labels.reshape(BATCH, 1)
    topk, x_cvs = _stage_a(x, cvs)
    pairs = topk.reshape(-1)
    cce = _make_sc_scatter()(pairs).reshape(NUM_CLASSES, NUM_CLASSES)
    gather = _make_sc_gather()
    half = BATCH // 2
    gath0 = gather(cce, labels[:half])
    gath1 = gather(cce, labels[half:])
    lossmat0 = _stage_e(x_cvs, gath0, labels2d, 0)
    lossmat1 = _stage_e(x_cvs, gath1, labels2d, 1)
    return (lossmat0[0, 0] + lossmat1[0, 0]) / jnp.float32(BATCH)
